# Initial kernel scaffold; baseline (speedup 1.0000x reference)
import jax
import jax.numpy as jnp
from jax.experimental import pallas as pl


def kernel(observations, W1, U1, b1, W2, U2, b2, Wm1, bm1, Wm2, bm2):
    raise NotImplementedError("write your pallas kernel here")


# speedup vs baseline: 2.1754x; 2.1754x over previous
import functools

import jax
import jax.numpy as jnp
from jax.experimental import pallas as pl
from jax.experimental.pallas import tpu as pltpu

DIM_STATE = 32
N_PARTICLES = 1024
DIM_OBS = 32
HIDDEN = 64
BATCH = 64
SEQ = 16

ROWS = BATCH * N_PARTICLES
BLK = 2048


def _step_kernel(x_ref, st_ref,
                 W1_ref, U1_ref, b1_ref, W2_ref, U2_ref, b2_ref,
                 Wm1_ref, bm1_ref, Wm2_ref, bm2_ref,
                 sto_ref, wo_ref):
    x = x_ref[...]
    st = st_ref[...]
    D = DIM_STATE

    def bdot(a, bmat):
        return jnp.dot(a.astype(jnp.bfloat16), bmat.astype(jnp.bfloat16),
                       preferred_element_type=jnp.float32)

    def lstm(xv, h, c, W, U, b):
        z = bdot(xv, W)
        z = z + bdot(h, U)
        z = z + b[None, :]
        i = jax.nn.sigmoid(z[:, 0:D])
        f = jax.nn.sigmoid(z[:, D:2 * D])
        g = jnp.tanh(z[:, 2 * D:3 * D])
        o = jax.nn.sigmoid(z[:, 3 * D:4 * D])
        c_new = f * c + i * g
        h_new = o * jnp.tanh(c_new)
        return h_new, c_new

    h1f, c1f = lstm(x, st[:, 0:D], st[:, D:2 * D], W1_ref[...], U1_ref[...], b1_ref[...][0])
    h2f, c2f = lstm(h1f, st[:, 2 * D:3 * D], st[:, 3 * D:4 * D],
                    W2_ref[...], U2_ref[...], b2_ref[...][0])
    sto_ref[...] = jnp.concatenate([h1f, c1f, h2f, c2f], axis=1)
    ob = x[:, D:D + DIM_OBS]
    minp = jnp.concatenate([ob, h2f], axis=1)
    hid = jnp.dot(minp.astype(jnp.bfloat16), Wm1_ref[...].astype(jnp.bfloat16),
                  preferred_element_type=jnp.float32)
    hid = jax.nn.relu(hid + bm1_ref[...][0])
    wv = jnp.dot(hid.astype(jnp.bfloat16), Wm2_ref[...].astype(jnp.bfloat16),
                 preferred_element_type=jnp.float32)
    wo_ref[...] = wv + bm2_ref[...][0, 0]


def _row_spec(width):
    return pl.BlockSpec((BLK, width), lambda i: (i, 0))


def _full_spec(shape):
    return pl.BlockSpec(shape, lambda i: tuple(0 for _ in shape))


def _step_pallas(x, st, W1, U1, b1, W2, U2, b2, Wm1, bm1, Wm2, bm2):
    n = ROWS // BLK
    out_shapes = [jax.ShapeDtypeStruct((ROWS, 4 * DIM_STATE), jnp.float32),
                  jax.ShapeDtypeStruct((ROWS, 1), jnp.float32)]
    in_specs = [
        _row_spec(DIM_STATE + DIM_OBS),
        _row_spec(4 * DIM_STATE),
        _full_spec(W1.shape), _full_spec(U1.shape), _full_spec((1, 4 * DIM_STATE)),
        _full_spec(W2.shape), _full_spec(U2.shape), _full_spec((1, 4 * DIM_STATE)),
        _full_spec(Wm1.shape), _full_spec((1, HIDDEN)),
        _full_spec(Wm2.shape), _full_spec((1, 1)),
    ]
    out_specs = [_row_spec(4 * DIM_STATE), _row_spec(1)]
    return pl.pallas_call(
        _step_kernel,
        grid=(n,),
        in_specs=in_specs,
        out_specs=out_specs,
        out_shape=out_shapes,
    )(x, st, W1, U1, b1.reshape(1, -1), W2, U2, b2.reshape(1, -1),
      Wm1, bm1.reshape(1, -1), Wm2, bm2.reshape(1, 1))


def kernel(observations, W1, U1, b1, W2, U2, b2, Wm1, bm1, Wm2, bm2):
    B, T, dim_obs = observations.shape
    P, D = N_PARTICLES, DIM_STATE
    h1 = jnp.zeros((B, P, D), jnp.float32)
    c1 = jnp.zeros((B, P, D), jnp.float32)
    h2 = jnp.zeros((B, P, D), jnp.float32)
    c2 = jnp.zeros((B, P, D), jnp.float32)
    w = jnp.ones((B, P), jnp.float32) / P
    keys = jax.random.split(jax.random.key(42), T)
    obs_t = jnp.transpose(observations, (1, 0, 2))

    def step(carry, xs):
        h1, c1, h2, c2, w = carry
        ob, key = xs
        k1, k2 = jax.random.split(key)
        idx = jax.random.categorical(k1, w, shape=(P, B)).T
        gi = idx[..., None]
        st = jnp.concatenate([h1, c1, h2, c2], axis=-1)
        st = jnp.take_along_axis(st, gi, axis=1)
        noise = jax.random.normal(k2, (B, P, D), jnp.float32)
        ob_t = jnp.broadcast_to(ob[:, None, :], (B, P, dim_obs))
        x = jnp.concatenate([noise, ob_t], axis=-1).reshape(B * P, D + dim_obs)
        sto, wv = _step_pallas(
            x, st.reshape(B * P, 4 * D),
            W1, U1, b1, W2, U2, b2, Wm1, bm1, Wm2, bm2)
        w_new = wv[:, 0].reshape(B, P)
        sto = sto.reshape(B, P, 4 * D)
        carry = (sto[..., 0:D], sto[..., D:2 * D], sto[..., 2 * D:3 * D],
                 sto[..., 3 * D:4 * D], w_new)
        return carry, None

    (h1, c1, h2, c2, w), _ = jax.lax.scan(step, (h1, c1, h2, c2, w), (obs_t, keys))
    return h2, w


# speedup vs baseline: 4.4647x; 2.0523x over previous
import functools

import jax
import jax.numpy as jnp
import numpy as np
from jax.experimental import pallas as pl
from jax.experimental.pallas import tpu as pltpu
from jax.sharding import PartitionSpec as P

DIM_STATE = 32
N_PARTICLES = 1024
DIM_OBS = 32
HIDDEN = 64
BATCH = 64
SEQ = 16

NDEV = 2
B_LOC = BATCH // NDEV
ROWS_L = B_LOC * N_PARTICLES
BLK = 2048

_TINY = np.float32(np.finfo(np.float32).tiny)
_LO = np.float32(np.nextafter(np.float32(-1.0), np.float32(0.0)))


def _threefry_xor(kd0, kd1, x1):
    ks0 = kd0
    ks1 = kd1
    ks2 = ks0 ^ ks1 ^ jnp.uint32(0x1BD11BDA)
    x0 = jnp.zeros_like(x1) + ks0
    x1 = x1 + ks1
    rots = ((13, 15, 26, 6), (17, 29, 16, 24))
    ks = (ks0, ks1, ks2)

    def rotl(x, d):
        return (x << jnp.uint32(d)) | (x >> jnp.uint32(32 - d))

    for i in range(5):
        for r in rots[i % 2]:
            x0 = x0 + x1
            x1 = rotl(x1, r)
            x1 = x0 ^ x1
        x0 = x0 + ks[(i + 1) % 3]
        x1 = x1 + ks[(i + 2) % 3] + jnp.uint32(i + 1)
    return x0 ^ x1


def _bits_to_unit(bits):
    fb = (bits >> jnp.uint32(9)) | jnp.uint32(0x3F800000)
    return jax.lax.bitcast_convert_type(fb, jnp.float32) - jnp.float32(1.0)


def _step_kernel(x_ref, st_ref,
                 W1_ref, U1_ref, b1_ref, W2_ref, U2_ref, b2_ref,
                 Wm1_ref, bm1_ref, Wm2_ref, bm2_ref,
                 sto_ref, wo_ref):
    x = x_ref[...]
    st = st_ref[...]
    D = DIM_STATE

    def bdot(a, bmat):
        return jnp.dot(a.astype(jnp.bfloat16), bmat.astype(jnp.bfloat16),
                       preferred_element_type=jnp.float32)

    def lstm(xv, h, c, W, U, b):
        z = bdot(xv, W)
        z = z + bdot(h, U)
        z = z + b[None, :]
        i = jax.nn.sigmoid(z[:, 0:D])
        f = jax.nn.sigmoid(z[:, D:2 * D])
        g = jnp.tanh(z[:, 2 * D:3 * D])
        o = jax.nn.sigmoid(z[:, 3 * D:4 * D])
        c_new = f * c + i * g
        h_new = o * jnp.tanh(c_new)
        return h_new, c_new

    h1f, c1f = lstm(x, st[:, 0:D], st[:, D:2 * D], W1_ref[...], U1_ref[...], b1_ref[...][0])
    h2f, c2f = lstm(h1f, st[:, 2 * D:3 * D], st[:, 3 * D:4 * D],
                    W2_ref[...], U2_ref[...], b2_ref[...][0])
    sto_ref[...] = jnp.concatenate([h1f, c1f, h2f, c2f], axis=1)
    ob = x[:, D:D + DIM_OBS]
    minp = jnp.concatenate([ob, h2f], axis=1)
    hid = jnp.dot(minp.astype(jnp.bfloat16), Wm1_ref[...].astype(jnp.bfloat16),
                  preferred_element_type=jnp.float32)
    hid = jax.nn.relu(hid + bm1_ref[...][0])
    wv = jnp.dot(hid.astype(jnp.bfloat16), Wm2_ref[...].astype(jnp.bfloat16),
                 preferred_element_type=jnp.float32)
    wo_ref[...] = wv + bm2_ref[...][0, 0]


def _row_spec(width):
    return pl.BlockSpec((BLK, width), lambda i: (i, 0))


def _full_spec(shape):
    return pl.BlockSpec(shape, lambda i: tuple(0 for _ in shape))


def _step_pallas(x, st, W1, U1, b1, W2, U2, b2, Wm1, bm1, Wm2, bm2):
    n = ROWS_L // BLK
    out_shapes = [jax.ShapeDtypeStruct((ROWS_L, 4 * DIM_STATE), jnp.float32),
                  jax.ShapeDtypeStruct((ROWS_L, 1), jnp.float32)]
    in_specs = [
        _row_spec(DIM_STATE + DIM_OBS),
        _row_spec(4 * DIM_STATE),
        _full_spec(W1.shape), _full_spec(U1.shape), _full_spec((1, 4 * DIM_STATE)),
        _full_spec(W2.shape), _full_spec(U2.shape), _full_spec((1, 4 * DIM_STATE)),
        _full_spec(Wm1.shape), _full_spec((1, HIDDEN)),
        _full_spec(Wm2.shape), _full_spec((1, 1)),
    ]
    out_specs = [_row_spec(4 * DIM_STATE), _row_spec(1)]
    return pl.pallas_call(
        _step_kernel,
        grid=(n,),
        in_specs=in_specs,
        out_specs=out_specs,
        out_shape=out_shapes,
    )(x, st, W1, U1, b1.reshape(1, -1), W2, U2, b2.reshape(1, -1),
      Wm1, bm1.reshape(1, -1), Wm2, bm2.reshape(1, 1))


def _shard_filter(obs_l, k1d, k2d, W1, U1, b1, W2, U2, b2, Wm1, bm1, Wm2, bm2):
    Pn, D, B = N_PARTICLES, DIM_STATE, BATCH
    b0 = jax.lax.axis_index("x") * B_LOC

    st = jnp.zeros((B_LOC, Pn, 4 * D), jnp.float32)
    w = jnp.ones((B_LOC, Pn), jnp.float32) / Pn
    obs_t = jnp.transpose(obs_l, (1, 0, 2))

    pp = jax.lax.broadcasted_iota(jnp.uint32, (Pn, B_LOC, Pn), 0)
    bb = jax.lax.broadcasted_iota(jnp.uint32, (Pn, B_LOC, Pn), 1)
    jj = jax.lax.broadcasted_iota(jnp.uint32, (Pn, B_LOC, Pn), 2)
    cat_idx = ((pp * jnp.uint32(B) + bb + b0.astype(jnp.uint32)) << jnp.uint32(10)) + jj

    np_idx = jax.lax.broadcasted_iota(jnp.uint32, (B_LOC, Pn, D), 0)
    np_p = jax.lax.broadcasted_iota(jnp.uint32, (B_LOC, Pn, D), 1)
    np_d = jax.lax.broadcasted_iota(jnp.uint32, (B_LOC, Pn, D), 2)
    noise_idx = (((np_idx + b0.astype(jnp.uint32)) * jnp.uint32(Pn) + np_p)
                 << jnp.uint32(5)) + np_d

    def step(carry, xs):
        st, w = carry
        ob, k1, k2 = xs
        bits = _threefry_xor(k1[0], k1[1], cat_idx)
        f = _bits_to_unit(bits)
        u = jnp.maximum(_TINY, f * (jnp.float32(1.0) - _TINY) + _TINY)
        g = -jnp.log(-jnp.log(u))
        idx = jnp.argmax(g + w[None, :, :], axis=-1).T
        st_g = jnp.take_along_axis(st, idx[..., None], axis=1)
        nbits = _threefry_xor(k2[0], k2[1], noise_idx)
        nf = _bits_to_unit(nbits)
        nu = jnp.maximum(_LO, nf * (jnp.float32(1.0) - _LO) + _LO)
        noise = jnp.sqrt(jnp.float32(2.0)) * jax.lax.erf_inv(nu)
        ob_t = jnp.broadcast_to(ob[:, None, :], (B_LOC, Pn, DIM_OBS))
        x = jnp.concatenate([noise, ob_t], axis=-1).reshape(ROWS_L, D + DIM_OBS)
        sto, wv = _step_pallas(x, st_g.reshape(ROWS_L, 4 * D),
                               W1, U1, b1, W2, U2, b2, Wm1, bm1, Wm2, bm2)
        w_new = wv[:, 0].reshape(B_LOC, Pn)
        return (sto.reshape(B_LOC, Pn, 4 * D), w_new), None

    (st, w), _ = jax.lax.scan(step, (st, w), (obs_t, k1d, k2d))
    return st[..., 2 * DIM_STATE:3 * DIM_STATE], w


def kernel(observations, W1, U1, b1, W2, U2, b2, Wm1, bm1, Wm2, bm2):
    T = SEQ
    keys = jax.random.split(jax.random.key(42), T)
    k12 = jax.vmap(jax.random.split)(keys)
    kd = jax.random.key_data(k12).astype(jnp.uint32)
    k1d, k2d = kd[:, 0, :], kd[:, 1, :]

    mesh = jax.make_mesh((NDEV,), ("x",))
    observations = jax.reshard(
        observations, jax.NamedSharding(mesh, P("x", None, None)))
    fn = jax.shard_map(
        _shard_filter, mesh=mesh,
        in_specs=(P("x"), P(), P(), P(), P(), P(), P(), P(), P(), P(), P(), P(), P()),
        out_specs=(P("x"), P("x")),
        check_vma=False,
    )
    return fn(observations, k1d, k2d, W1, U1, b1, W2, U2, b2, Wm1, bm1, Wm2, bm2)
